# Initial kernel scaffold; baseline (speedup 1.0000x reference)
import jax
import jax.numpy as jnp
from jax.experimental import pallas as pl


def kernel(z):
    raise NotImplementedError("write your pallas kernel here")


# speedup vs baseline: 15.2402x; 15.2402x over previous
import functools

import jax
import jax.numpy as jnp
from jax import lax
from jax.experimental import pallas as pl
from jax.experimental.pallas import tpu as pltpu
from jax.experimental.pallas import tpu_sc as plsc

B = 64
N = 32768
NC = 2
NS = 16
L = 16
NW = NC * NS
ROWS_PER_W = B // NW
NVEC = N // L

_NEG = -3.0e38


def _row_sparsemax(row_v, cand_v):
    def mx_body(i, acc):
        return jnp.maximum(acc, row_v[pl.ds(i * L, L)])

    mvec = lax.fori_loop(0, NVEC, mx_body, jnp.full((L,), _NEG, jnp.float32))
    m = jnp.max(mvec)
    thr = m - jnp.float32(1.0)

    def cp_body(i, off):
        v = row_v[pl.ds(i * L, L)]
        msk = v >= thr
        plsc.store_compressed(cand_v.at[pl.ds(off, L)], v, mask=msk)
        return off + jnp.max(plsc.all_reduce_population_count(msk))

    ncand = lax.fori_loop(0, NVEC, cp_body, jnp.int32(0))
    cand_v[pl.ds(ncand, L)] = jnp.full((L,), _NEG, jnp.float32)
    nv = (ncand + (L - 1)) >> 4

    def tau_from(sel_thresh_fn):
        def body(i, acc):
            s, c = acc
            v = cand_v[pl.ds(i * L, L)]
            sel = sel_thresh_fn(v)
            return (s + jnp.where(sel, v, jnp.float32(0.0)),
                    c + jnp.where(sel, jnp.int32(1), jnp.int32(0)))

        s, c = lax.fori_loop(
            0, nv, body,
            (jnp.zeros((L,), jnp.float32), jnp.zeros((L,), jnp.int32)))
        cs = jnp.sum(c)
        sv = jnp.full((L,), jnp.sum(s))
        cv = jnp.full((L,), cs).astype(jnp.float32)
        return (sv - jnp.float32(1.0)) / cv, cs

    t0, _ = tau_from(lambda v: v >= m)

    def w_cond(st):
        _, cprev, cnow, it = st
        return jnp.logical_and(cnow != cprev, it < jnp.int32(128))

    def w_body(st):
        t, _, cnow, it = st
        t2, c = tau_from(lambda v: v > t)
        return (t2, cnow, c, it + jnp.int32(1))

    tau, _, _, _ = lax.while_loop(
        w_cond, w_body, (t0, jnp.int32(-1), jnp.int32(-2), jnp.int32(0)))

    def out_body(i, _):
        v = row_v[pl.ds(i * L, L)]
        row_v[pl.ds(i * L, L)] = jnp.maximum(v - tau, jnp.float32(0.0))
        return 0

    lax.fori_loop(0, NVEC, out_body, 0)


@functools.partial(
    pl.kernel,
    out_type=jax.ShapeDtypeStruct((B, N), jnp.float32),
    mesh=plsc.VectorSubcoreMesh(core_axis_name="c", subcore_axis_name="s"),
    compiler_params=pltpu.CompilerParams(needs_layout_passes=False),
    scratch_types=[
        pltpu.VMEM((N,), jnp.float32),
        pltpu.VMEM((N + L,), jnp.float32),
    ],
)
def _sparsemax_sc(z_hbm, out_hbm, row_v, cand_v):
    wid = lax.axis_index("s") * NC + lax.axis_index("c")
    for r in range(ROWS_PER_W):
        row = wid * ROWS_PER_W + r
        pltpu.sync_copy(z_hbm.at[row], row_v)
        _row_sparsemax(row_v, cand_v)
        pltpu.sync_copy(row_v, out_hbm.at[row])


def kernel(z):
    assert z.shape == (B, N) and z.dtype == jnp.float32
    return _sparsemax_sc(z)


# speedup vs baseline: 17.9935x; 1.1807x over previous
import functools

import jax
import jax.numpy as jnp
from jax import lax
from jax.experimental import pallas as pl
from jax.experimental.pallas import tpu as pltpu
from jax.experimental.pallas import tpu_sc as plsc

B = 64
N = 32768
NC = 2
NS = 16
L = 16
NW = NC * NS
ROWS_PER_W = B // NW
NVEC = N // L
U = 8

_NEG = -3.0e38


def _splat_f(x):
    return jnp.full((L,), x, jnp.float32)


def _row_tau(row_v, cand_v):
    ones = jnp.full((L,), 1, jnp.int32)
    zeros = jnp.full((L,), 0, jnp.int32)

    def sweep_body(i, carry):
        acc, off = carry
        for u in range(U):
            v = row_v[pl.ds((i + u) * L, L)]
            acc = jnp.maximum(acc, v)
            msk = v >= acc - jnp.float32(1.0)
            pos = plsc.cumsum(jnp.where(msk, ones, zeros))
            idx = jnp.maximum(off + pos - 1, 0)
            plsc.store_scatter(cand_v, [idx], v, mask=msk)
            off = off + plsc.all_reduce_population_count(msk)
        return acc, off

    acc, off_v = lax.fori_loop(
        0, NVEC // U, lambda i, c: sweep_body(i * U, c),
        (_splat_f(_NEG), zeros))

    m = jnp.max(acc)
    ncand = jnp.max(off_v)
    iota = lax.iota(jnp.int32, L)
    plsc.store_scatter(cand_v, [off_v + iota], _splat_f(_NEG))
    nv = (ncand + (L - 1)) >> 4

    def tau_from(sel_fn):
        def body(i, acc2):
            s, c = acc2
            v = cand_v[pl.ds(i * L, L)]
            sel = sel_fn(v)
            return (s + jnp.where(sel, v, jnp.float32(0.0)),
                    c + jnp.where(sel, ones, zeros))

        s, c = lax.fori_loop(0, nv, body, (_splat_f(0.0), zeros))
        cs = jnp.sum(c)
        sv = _splat_f(jnp.sum(s))
        cv = jnp.full((L,), cs).astype(jnp.float32)
        return (sv - jnp.float32(1.0)) / cv, cs

    t0, _ = tau_from(lambda v: v >= m)

    def w_cond(st):
        _, cprev, cnow, it = st
        return jnp.logical_and(cnow != cprev, it < jnp.int32(128))

    def w_body(st):
        t, _, cnow, it = st
        t2, c = tau_from(lambda v: v > t)
        return (t2, cnow, c, it + jnp.int32(1))

    tau, _, _, _ = lax.while_loop(
        w_cond, w_body, (t0, jnp.int32(-1), jnp.int32(-2), jnp.int32(0)))
    return tau


def _row_out(row_v, tau):
    def out_body(i):
        for u in range(U):
            sl = pl.ds((i + u) * L, L)
            row_v[sl] = jnp.maximum(row_v[sl] - tau, jnp.float32(0.0))

    plsc.parallel_loop(0, NVEC, U)(out_body)


@functools.partial(
    pl.kernel,
    out_type=jax.ShapeDtypeStruct((B, N), jnp.float32),
    mesh=plsc.VectorSubcoreMesh(core_axis_name="c", subcore_axis_name="s"),
    compiler_params=pltpu.CompilerParams(needs_layout_passes=False),
    scratch_types=[
        pltpu.VMEM((N,), jnp.float32),
        pltpu.VMEM((N,), jnp.float32),
        pltpu.VMEM((N + L,), jnp.float32),
        pltpu.SemaphoreType.DMA,
        pltpu.SemaphoreType.DMA,
        pltpu.SemaphoreType.DMA,
        pltpu.SemaphoreType.DMA,
    ],
)
def _sparsemax_sc(z_hbm, out_hbm, row0_v, row1_v, cand_v,
                  in0_sem, in1_sem, out0_sem, out1_sem):
    wid = lax.axis_index("s") * NC + lax.axis_index("c")
    r0 = wid * ROWS_PER_W
    r1 = r0 + 1
    in0 = pltpu.make_async_copy(z_hbm.at[r0], row0_v, in0_sem)
    in1 = pltpu.make_async_copy(z_hbm.at[r1], row1_v, in1_sem)
    in0.start()
    in1.start()
    in0.wait()
    tau0 = _row_tau(row0_v, cand_v)
    _row_out(row0_v, tau0)
    out0 = pltpu.make_async_copy(row0_v, out_hbm.at[r0], out0_sem)
    out0.start()
    in1.wait()
    tau1 = _row_tau(row1_v, cand_v)
    _row_out(row1_v, tau1)
    out1 = pltpu.make_async_copy(row1_v, out_hbm.at[r1], out1_sem)
    out1.start()
    out0.wait()
    out1.wait()


def kernel(z):
    assert z.shape == (B, N) and z.dtype == jnp.float32
    return _sparsemax_sc(z)


# speedup vs baseline: 24.0751x; 1.3380x over previous
import functools

import jax
import jax.numpy as jnp
from jax import lax
from jax.experimental import pallas as pl
from jax.experimental.pallas import tpu as pltpu
from jax.experimental.pallas import tpu_sc as plsc

B = 64
N = 32768
NC = 2
NS = 16
L = 16
NW = NC * NS
ROWS_PER_W = B // NW
NVEC = N // L
U = 8

_NEG = -3.0e38


def _splat_f(x):
    return jnp.full((L,), x, jnp.float32)


def _row_tau(row_v, cand_v):
    ones = jnp.full((L,), 1, jnp.int32)
    zeros = jnp.full((L,), 0, jnp.int32)
    iota = lax.iota(jnp.int32, L)

    def sweep_body(i, carry):
        acc, idx, cnt = carry
        for u in range(U):
            v = row_v[pl.ds((i * U + u) * L, L)]
            acc = jnp.maximum(acc, v)
            msk = v >= acc - jnp.float32(1.0)
            plsc.store_scatter(cand_v, [idx], v, mask=msk)
            bump = jnp.where(msk, jnp.int32(L), jnp.int32(0))
            idx = idx + bump
            cnt = cnt + jnp.where(msk, ones, zeros)
        return acc, idx, cnt

    acc, _, cnt_v = lax.fori_loop(
        0, NVEC // U, sweep_body, (_splat_f(_NEG), iota, zeros))

    m = jnp.max(acc)
    nv = jnp.max(cnt_v)

    def tau_from(sel_fn):
        def body(j, acc2):
            s, c = acc2
            v = cand_v[pl.ds(j * L, L)]
            sel = jnp.logical_and(cnt_v > j, sel_fn(v))
            return (s + jnp.where(sel, v, jnp.float32(0.0)),
                    c + jnp.where(sel, ones, zeros))

        s, c = lax.fori_loop(0, nv, body, (_splat_f(0.0), zeros))
        cs = jnp.sum(c)
        sv = _splat_f(jnp.sum(s))
        cv = jnp.full((L,), cs).astype(jnp.float32)
        return (sv - jnp.float32(1.0)) / cv, cs

    t0, _ = tau_from(lambda v: v >= m)

    def w_cond(st):
        _, cprev, cnow, it = st
        return jnp.logical_and(cnow != cprev, it < jnp.int32(128))

    def w_body(st):
        t, _, cnow, it = st
        t2, c = tau_from(lambda v: v > t)
        return (t2, cnow, c, it + jnp.int32(1))

    tau, _, _, _ = lax.while_loop(
        w_cond, w_body, (t0, jnp.int32(-1), jnp.int32(-2), jnp.int32(0)))
    return tau


def _row_out(row_v, tau):
    def out_body(i, _):
        for u in range(U):
            sl = pl.ds((i * U + u) * L, L)
            row_v[sl] = jnp.maximum(row_v[sl] - tau, jnp.float32(0.0))
        return 0

    lax.fori_loop(0, NVEC // U, out_body, 0)


@functools.partial(
    pl.kernel,
    out_type=jax.ShapeDtypeStruct((B, N), jnp.float32),
    mesh=plsc.VectorSubcoreMesh(core_axis_name="c", subcore_axis_name="s"),
    compiler_params=pltpu.CompilerParams(needs_layout_passes=False),
    scratch_types=[
        pltpu.VMEM((N,), jnp.float32),
        pltpu.VMEM((N,), jnp.float32),
        pltpu.VMEM((N,), jnp.float32),
        pltpu.SemaphoreType.DMA,
        pltpu.SemaphoreType.DMA,
        pltpu.SemaphoreType.DMA,
        pltpu.SemaphoreType.DMA,
    ],
)
def _sparsemax_sc(z_hbm, out_hbm, row0_v, row1_v, cand_v,
                  in0_sem, in1_sem, out0_sem, out1_sem):
    wid = lax.axis_index("s") * NC + lax.axis_index("c")
    r0 = wid * ROWS_PER_W
    r1 = r0 + 1
    in0 = pltpu.make_async_copy(z_hbm.at[r0], row0_v, in0_sem)
    in1 = pltpu.make_async_copy(z_hbm.at[r1], row1_v, in1_sem)
    in0.start()
    in1.start()
    in0.wait()
    tau0 = _row_tau(row0_v, cand_v)
    _row_out(row0_v, tau0)
    out0 = pltpu.make_async_copy(row0_v, out_hbm.at[r0], out0_sem)
    out0.start()
    in1.wait()
    tau1 = _row_tau(row1_v, cand_v)
    _row_out(row1_v, tau1)
    out1 = pltpu.make_async_copy(row1_v, out_hbm.at[r1], out1_sem)
    out1.start()
    out0.wait()
    out1.wait()


def kernel(z):
    assert z.shape == (B, N) and z.dtype == jnp.float32
    return _sparsemax_sc(z)


# speedup vs baseline: 26.8220x; 1.1141x over previous
import functools

import jax
import jax.numpy as jnp
from jax import lax
from jax.experimental import pallas as pl
from jax.experimental.pallas import tpu as pltpu
from jax.experimental.pallas import tpu_sc as plsc

B = 64
N = 32768
NC = 2
NS = 16
L = 16
NW = NC * NS
ROWS_PER_W = B // NW
NVEC = N // L
U = 8

_NEG = -3.0e38


def _splat_f(x):
    return jnp.full((L,), x, jnp.float32)


def _row_tau(row_v, cand_v):
    ones = jnp.full((L,), 1, jnp.int32)
    zeros = jnp.full((L,), 0, jnp.int32)
    iota = lax.iota(jnp.int32, L)

    def sweep_body(i, carry):
        thp, acc, idx = carry
        for u in range(U):
            v = row_v[pl.ds((i * U + u) * L, L)]
            msk = v >= thp
            plsc.store_scatter(cand_v, [idx], v, mask=msk)
            idx = idx + jnp.where(msk, jnp.int32(L), jnp.int32(0))
            thp = jnp.maximum(thp, v - jnp.float32(1.0))
            acc = jnp.maximum(acc, v)
        return thp, acc, idx

    _, acc, idx_v = lax.fori_loop(
        0, NVEC // U, sweep_body, (_splat_f(_NEG), _splat_f(_NEG), iota))
    cnt_v = jnp.right_shift(idx_v - iota, 4)

    m = jnp.max(acc)
    nv = jnp.max(cnt_v)

    def tau_from(sel_fn):
        def body(j, acc2):
            s, c = acc2
            v = cand_v[pl.ds(j * L, L)]
            sel = jnp.logical_and(cnt_v > j, sel_fn(v))
            return (s + jnp.where(sel, v, jnp.float32(0.0)),
                    c + jnp.where(sel, ones, zeros))

        s, c = lax.fori_loop(0, nv, body, (_splat_f(0.0), zeros))
        cs = jnp.sum(c)
        sv = _splat_f(jnp.sum(s))
        cv = jnp.full((L,), cs).astype(jnp.float32)
        return (sv - jnp.float32(1.0)) / cv, cs

    t0, _ = tau_from(lambda v: v >= m)

    def w_cond(st):
        _, cprev, cnow, it = st
        return jnp.logical_and(cnow != cprev, it < jnp.int32(128))

    def w_body(st):
        t, _, cnow, it = st
        t2, c = tau_from(lambda v: v > t)
        return (t2, cnow, c, it + jnp.int32(1))

    tau, _, _, _ = lax.while_loop(
        w_cond, w_body, (t0, jnp.int32(-1), jnp.int32(-2), jnp.int32(0)))
    return tau


def _row_out(row_v, tau):
    def out_body(i, _):
        for u in range(U):
            sl = pl.ds((i * U + u) * L, L)
            row_v[sl] = jnp.maximum(row_v[sl] - tau, jnp.float32(0.0))
        return 0

    lax.fori_loop(0, NVEC // U, out_body, 0)


@functools.partial(
    pl.kernel,
    out_type=jax.ShapeDtypeStruct((B, N), jnp.float32),
    mesh=plsc.VectorSubcoreMesh(core_axis_name="c", subcore_axis_name="s"),
    compiler_params=pltpu.CompilerParams(needs_layout_passes=False),
    scratch_types=[
        pltpu.VMEM((N,), jnp.float32),
        pltpu.VMEM((N,), jnp.float32),
        pltpu.VMEM((N,), jnp.float32),
        pltpu.SemaphoreType.DMA,
        pltpu.SemaphoreType.DMA,
        pltpu.SemaphoreType.DMA,
        pltpu.SemaphoreType.DMA,
    ],
)
def _sparsemax_sc(z_hbm, out_hbm, row0_v, row1_v, cand_v,
                  in0_sem, in1_sem, out0_sem, out1_sem):
    wid = lax.axis_index("s") * NC + lax.axis_index("c")
    r0 = wid * ROWS_PER_W
    r1 = r0 + 1
    in0 = pltpu.make_async_copy(z_hbm.at[r0], row0_v, in0_sem)
    in1 = pltpu.make_async_copy(z_hbm.at[r1], row1_v, in1_sem)
    in0.start()
    in1.start()
    in0.wait()
    tau0 = _row_tau(row0_v, cand_v)
    _row_out(row0_v, tau0)
    out0 = pltpu.make_async_copy(row0_v, out_hbm.at[r0], out0_sem)
    out0.start()
    in1.wait()
    tau1 = _row_tau(row1_v, cand_v)
    _row_out(row1_v, tau1)
    out1 = pltpu.make_async_copy(row1_v, out_hbm.at[r1], out1_sem)
    out1.start()
    out0.wait()
    out1.wait()


def kernel(z):
    assert z.shape == (B, N) and z.dtype == jnp.float32
    return _sparsemax_sc(z)


# speedup vs baseline: 44.9156x; 1.6746x over previous
import functools

import jax
import jax.numpy as jnp
from jax import lax
from jax.experimental import pallas as pl
from jax.experimental.pallas import tpu as pltpu
from jax.experimental.pallas import tpu_sc as plsc

B = 64
N = 32768
NC = 2
NS = 16
L = 16
NW = NC * NS
ROWS_PER_W = B // NW
NVEC = N // L
U = 8

_NEG = -3.0e38


def _splat_f(x):
    return jnp.full((L,), x, jnp.float32)


def _row_tau(row_v, cand_v):
    ones = jnp.full((L,), 1, jnp.int32)
    zeros = jnp.full((L,), 0, jnp.int32)
    iota = lax.iota(jnp.int32, L)

    RU = N // U

    def sweep_body(i, carry):
        thp, acc, idxs = carry
        vs = [row_v[pl.ds((i * U + u) * L, L)] for u in range(U)]
        new_idxs = []
        for u in range(U):
            msk = vs[u] >= thp
            plsc.store_scatter(cand_v, [idxs[u]], vs[u], mask=msk)
            new_idxs.append(
                idxs[u] + jnp.where(msk, jnp.int32(L), jnp.int32(0)))
        bmax = vs
        while len(bmax) > 1:
            bmax = [jnp.maximum(a, b) for a, b in zip(bmax[::2], bmax[1::2])]
        acc = jnp.maximum(acc, bmax[0])
        thp = jnp.maximum(thp, bmax[0] - jnp.float32(1.0))
        return thp, acc, tuple(new_idxs)

    idxs0 = tuple(jnp.full((L,), u * RU, jnp.int32) + iota for u in range(U))
    _, acc, idxs_v = lax.fori_loop(
        0, NVEC // U, sweep_body, (_splat_f(_NEG), _splat_f(_NEG), idxs0))
    cnts = [jnp.right_shift(idxs_v[u] - iota, 4) - jnp.int32(u * (RU >> 4))
            for u in range(U)]

    m = jnp.max(acc)
    cmax = cnts
    while len(cmax) > 1:
        cmax = [jnp.maximum(a, b) for a, b in zip(cmax[::2], cmax[1::2])]
    nv = jnp.max(cmax[0])

    def tau_from(sel_fn):
        def body(j, acc2):
            s, c = acc2
            for u in range(U):
                v = cand_v[pl.ds(u * RU + j * L, L)]
                sel = jnp.logical_and(cnts[u] > j, sel_fn(v))
                s = s + jnp.where(sel, v, jnp.float32(0.0))
                c = c + jnp.where(sel, ones, zeros)
            return s, c

        s, c = lax.fori_loop(0, nv, body, (_splat_f(0.0), zeros))
        cs = jnp.sum(c)
        sv = _splat_f(jnp.sum(s))
        cv = jnp.full((L,), cs).astype(jnp.float32)
        return (sv - jnp.float32(1.0)) / cv, cs

    t0, _ = tau_from(lambda v: v >= m)

    def w_cond(st):
        _, cprev, cnow, it = st
        return jnp.logical_and(cnow != cprev, it < jnp.int32(128))

    def w_body(st):
        t, _, cnow, it = st
        t2, c = tau_from(lambda v: v > t)
        return (t2, cnow, c, it + jnp.int32(1))

    tau, _, _, _ = lax.while_loop(
        w_cond, w_body, (t0, jnp.int32(-1), jnp.int32(-2), jnp.int32(0)))
    return tau


def _row_out(row_v, tau):
    def out_body(i, _):
        for u in range(U):
            sl = pl.ds((i * U + u) * L, L)
            row_v[sl] = jnp.maximum(row_v[sl] - tau, jnp.float32(0.0))
        return 0

    lax.fori_loop(0, NVEC // U, out_body, 0)


@functools.partial(
    pl.kernel,
    out_type=jax.ShapeDtypeStruct((B, N), jnp.float32),
    mesh=plsc.VectorSubcoreMesh(core_axis_name="c", subcore_axis_name="s"),
    compiler_params=pltpu.CompilerParams(needs_layout_passes=False),
    scratch_types=[
        pltpu.VMEM((N,), jnp.float32),
        pltpu.VMEM((N,), jnp.float32),
        pltpu.VMEM((N,), jnp.float32),
        pltpu.SemaphoreType.DMA,
        pltpu.SemaphoreType.DMA,
        pltpu.SemaphoreType.DMA,
        pltpu.SemaphoreType.DMA,
    ],
)
def _sparsemax_sc(z_hbm, out_hbm, row0_v, row1_v, cand_v,
                  in0_sem, in1_sem, out0_sem, out1_sem):
    wid = lax.axis_index("s") * NC + lax.axis_index("c")
    r0 = wid * ROWS_PER_W
    r1 = r0 + 1
    in0 = pltpu.make_async_copy(z_hbm.at[r0], row0_v, in0_sem)
    in1 = pltpu.make_async_copy(z_hbm.at[r1], row1_v, in1_sem)
    in0.start()
    in1.start()
    in0.wait()
    tau0 = _row_tau(row0_v, cand_v)
    _row_out(row0_v, tau0)
    out0 = pltpu.make_async_copy(row0_v, out_hbm.at[r0], out0_sem)
    out0.start()
    in1.wait()
    tau1 = _row_tau(row1_v, cand_v)
    _row_out(row1_v, tau1)
    out1 = pltpu.make_async_copy(row1_v, out_hbm.at[r1], out1_sem)
    out1.start()
    out0.wait()
    out1.wait()


def kernel(z):
    assert z.shape == (B, N) and z.dtype == jnp.float32
    return _sparsemax_sc(z)


# speedup vs baseline: 44.9507x; 1.0008x over previous
import functools

import jax
import jax.numpy as jnp
from jax import lax
from jax.experimental import pallas as pl
from jax.experimental.pallas import tpu as pltpu
from jax.experimental.pallas import tpu_sc as plsc

B = 64
N = 32768
NC = 2
NS = 16
L = 16
NW = NC * NS
ROWS_PER_W = B // NW
NVEC = N // L
U = 8

_NEG = -3.0e38


def _splat_f(x):
    return jnp.full((L,), x, jnp.float32)


def _row_tau(row_v, cand_v):
    ones = jnp.full((L,), 1, jnp.int32)
    zeros = jnp.full((L,), 0, jnp.int32)
    iota = lax.iota(jnp.int32, L)

    RU = N // U

    def sweep_body(i, carry):
        thp, acc, idxs = carry
        vs = [row_v[pl.ds((i * U + u) * L, L)] for u in range(U)]
        new_idxs = []
        for u in range(U):
            msk = vs[u] >= thp
            plsc.store_scatter(cand_v, [idxs[u]], vs[u], mask=msk)
            new_idxs.append(
                idxs[u] + jnp.where(msk, jnp.int32(L), jnp.int32(0)))
        bmax = vs
        while len(bmax) > 1:
            bmax = [jnp.maximum(a, b) for a, b in zip(bmax[::2], bmax[1::2])]
        acc = jnp.maximum(acc, bmax[0])
        thp = jnp.maximum(thp, bmax[0] - jnp.float32(1.0))
        return thp, acc, tuple(new_idxs)

    idxs0 = tuple(jnp.full((L,), u * RU, jnp.int32) + iota for u in range(U))
    _, acc, idxs_v = lax.fori_loop(
        0, NVEC // U, sweep_body, (_splat_f(_NEG), _splat_f(_NEG), idxs0))
    cnts = [jnp.right_shift(idxs_v[u] - iota, 4) - jnp.int32(u * (RU >> 4))
            for u in range(U)]

    m = jnp.max(acc)
    cmax = cnts
    while len(cmax) > 1:
        cmax = [jnp.maximum(a, b) for a, b in zip(cmax[::2], cmax[1::2])]
    nv = jnp.max(cmax[0])

    def tau_from(sel_fn):
        def body(j, acc2):
            s, c = acc2
            for u in range(U):
                v = cand_v[pl.ds(u * RU + j * L, L)]
                sel = jnp.logical_and(cnts[u] > j, sel_fn(v))
                s = s + jnp.where(sel, v, jnp.float32(0.0))
                c = c + jnp.where(sel, ones, zeros)
            return s, c

        s, c = lax.fori_loop(0, nv, body, (_splat_f(0.0), zeros))
        cs = jnp.sum(c)
        sv = _splat_f(jnp.sum(s))
        cv = jnp.full((L,), cs).astype(jnp.float32)
        return (sv - jnp.float32(1.0)) / cv, cs

    t0, _ = tau_from(lambda v: v >= m)

    def w_cond(st):
        _, cprev, cnow, it = st
        return jnp.logical_and(cnow != cprev, it < jnp.int32(128))

    def w_body(st):
        t, _, cnow, it = st
        t2, c = tau_from(lambda v: v > t)
        return (t2, cnow, c, it + jnp.int32(1))

    tau, _, _, _ = lax.while_loop(
        w_cond, w_body, (t0, jnp.int32(-1), jnp.int32(-2), jnp.int32(0)))
    return tau


def _row_out(row_v, tau):
    def out_body(i, _):
        for u in range(U):
            sl = pl.ds((i * U + u) * L, L)
            row_v[sl] = jnp.maximum(row_v[sl] - tau, jnp.float32(0.0))
        return 0

    lax.fori_loop(0, NVEC // U, out_body, 0)


@functools.partial(
    pl.kernel,
    out_type=jax.ShapeDtypeStruct((B, N), jnp.float32),
    mesh=plsc.VectorSubcoreMesh(core_axis_name="c", subcore_axis_name="s"),
    compiler_params=pltpu.CompilerParams(
        needs_layout_passes=False, skip_device_barrier=True),
    scratch_types=[
        pltpu.VMEM((N,), jnp.float32),
        pltpu.VMEM((N,), jnp.float32),
        pltpu.VMEM((N,), jnp.float32),
        pltpu.SemaphoreType.DMA,
        pltpu.SemaphoreType.DMA,
        pltpu.SemaphoreType.DMA,
        pltpu.SemaphoreType.DMA,
    ],
)
def _sparsemax_sc(z_hbm, out_hbm, row0_v, row1_v, cand_v,
                  in0_sem, in1_sem, out0_sem, out1_sem):
    wid = lax.axis_index("s") * NC + lax.axis_index("c")
    r0 = wid * ROWS_PER_W
    r1 = r0 + 1
    in0 = pltpu.make_async_copy(z_hbm.at[r0], row0_v, in0_sem)
    in1 = pltpu.make_async_copy(z_hbm.at[r1], row1_v, in1_sem)
    in0.start()
    in1.start()
    in0.wait()
    tau0 = _row_tau(row0_v, cand_v)
    _row_out(row0_v, tau0)
    out0 = pltpu.make_async_copy(row0_v, out_hbm.at[r0], out0_sem)
    out0.start()
    in1.wait()
    tau1 = _row_tau(row1_v, cand_v)
    _row_out(row1_v, tau1)
    out1 = pltpu.make_async_copy(row1_v, out_hbm.at[r1], out1_sem)
    out1.start()
    out0.wait()
    out1.wait()


def kernel(z):
    assert z.shape == (B, N) and z.dtype == jnp.float32
    return _sparsemax_sc(z)


# speedup vs baseline: 45.0208x; 1.0016x over previous
import functools

import jax
import jax.numpy as jnp
from jax import lax
from jax.experimental import pallas as pl
from jax.experimental.pallas import tpu as pltpu
from jax.experimental.pallas import tpu_sc as plsc

B = 64
N = 32768
NC = 2
NS = 16
L = 16
NW = NC * NS
ROWS_PER_W = B // NW
NVEC = N // L
U = 8

_NEG = -3.0e38


def _splat_f(x):
    return jnp.full((L,), x, jnp.float32)


def _row_tau(row_v, cand_v):
    ones = jnp.full((L,), 1, jnp.int32)
    zeros = jnp.full((L,), 0, jnp.int32)
    iota = lax.iota(jnp.int32, L)

    RU = N // U

    def sweep_body(i, carry):
        thp, acc, idxs = carry
        vs = [row_v[pl.ds((i * U + u) * L, L)] for u in range(U)]
        new_idxs = []
        for u in range(U):
            msk = vs[u] >= thp
            plsc.store_scatter(cand_v, [idxs[u]], vs[u], mask=msk)
            new_idxs.append(
                idxs[u] + jnp.where(msk, jnp.int32(L), jnp.int32(0)))
        bmax = vs
        while len(bmax) > 1:
            bmax = [jnp.maximum(a, b) for a, b in zip(bmax[::2], bmax[1::2])]
        acc = jnp.maximum(acc, bmax[0])
        thp = jnp.maximum(thp, bmax[0] - jnp.float32(1.0))
        return thp, acc, tuple(new_idxs)

    idxs0 = tuple(jnp.full((L,), u * RU, jnp.int32) + iota for u in range(U))
    _, acc, idxs_v = lax.fori_loop(
        0, NVEC // U, sweep_body, (_splat_f(_NEG), _splat_f(_NEG), idxs0))
    cnts = [jnp.right_shift(idxs_v[u] - iota, 4) - jnp.int32(u * (RU >> 4))
            for u in range(U)]

    m = jnp.max(acc)
    cmax = cnts
    while len(cmax) > 1:
        cmax = [jnp.maximum(a, b) for a, b in zip(cmax[::2], cmax[1::2])]
    nv = jnp.max(cmax[0])

    def tau_from(sel_fn):
        def body(j, acc2):
            s, c = acc2
            for u in range(U):
                v = cand_v[pl.ds(u * RU + j * L, L)]
                sel = jnp.logical_and(cnts[u] > j, sel_fn(v))
                s = s + jnp.where(sel, v, jnp.float32(0.0))
                c = c + jnp.where(sel, ones, zeros)
            return s, c

        s, c = lax.fori_loop(0, nv, body, (_splat_f(0.0), zeros))
        cs = jnp.sum(c)
        sv = _splat_f(jnp.sum(s))
        cv = jnp.full((L,), cs).astype(jnp.float32)
        return (sv - jnp.float32(1.0)) / cv, cs

    t0, _ = tau_from(lambda v: v >= m)

    def w_cond(st):
        _, cprev, cnow, it = st
        return jnp.logical_and(cnow != cprev, it < jnp.int32(128))

    def w_body(st):
        t, _, cnow, it = st
        t2, c = tau_from(lambda v: v > t)
        return (t2, cnow, c, it + jnp.int32(1))

    tau, _, _, _ = lax.while_loop(
        w_cond, w_body, (t0, jnp.int32(-1), jnp.int32(-2), jnp.int32(0)))
    return tau


def _row_out(row_v, tau):
    def out_body(i, _):
        for u in range(U):
            sl = pl.ds((i * U + u) * L, L)
            row_v[sl] = jnp.maximum(row_v[sl] - tau, jnp.float32(0.0))
        return 0

    lax.fori_loop(0, NVEC // U, out_body, 0)


@functools.partial(
    pl.kernel,
    out_type=jax.ShapeDtypeStruct((B, N), jnp.float32),
    mesh=plsc.VectorSubcoreMesh(core_axis_name="c", subcore_axis_name="s"),
    compiler_params=pltpu.CompilerParams(needs_layout_passes=False),
    scratch_types=[
        pltpu.VMEM((N,), jnp.float32),
        pltpu.VMEM((N,), jnp.float32),
        pltpu.VMEM((N,), jnp.float32),
        pltpu.SemaphoreType.DMA,
        pltpu.SemaphoreType.DMA,
        pltpu.SemaphoreType.DMA,
        pltpu.SemaphoreType.DMA,
    ],
)
def _sparsemax_sc(z_hbm, out_hbm, row0_v, row1_v, cand_v,
                  in0_sem, in1_sem, out0_sem, out1_sem):
    wid = lax.axis_index("s") * NC + lax.axis_index("c")
    r0 = wid * ROWS_PER_W
    r1 = r0 + 1
    in0 = pltpu.make_async_copy(z_hbm.at[r0], row0_v, in0_sem)
    in1 = pltpu.make_async_copy(z_hbm.at[r1], row1_v, in1_sem)
    in0.start()
    in1.start()
    with jax.named_scope("in0_wait"):
        in0.wait()
    with jax.named_scope("tau0"):
        tau0 = _row_tau(row0_v, cand_v)
    with jax.named_scope("out0"):
        _row_out(row0_v, tau0)
    out0 = pltpu.make_async_copy(row0_v, out_hbm.at[r0], out0_sem)
    out0.start()
    with jax.named_scope("in1_wait"):
        in1.wait()
    with jax.named_scope("tau1"):
        tau1 = _row_tau(row1_v, cand_v)
    with jax.named_scope("out1"):
        _row_out(row1_v, tau1)
    out1 = pltpu.make_async_copy(row1_v, out_hbm.at[r1], out1_sem)
    out1.start()
    with jax.named_scope("drain"):
        out0.wait()
        out1.wait()


def kernel(z):
    assert z.shape == (B, N) and z.dtype == jnp.float32
    return _sparsemax_sc(z)


# speedup vs baseline: 46.9422x; 1.0427x over previous
import functools

import jax
import jax.numpy as jnp
from jax import lax
from jax.experimental import pallas as pl
from jax.experimental.pallas import tpu as pltpu
from jax.experimental.pallas import tpu_sc as plsc

B = 64
N = 32768
NC = 2
NS = 16
L = 16
NW = NC * NS
ROWS_PER_W = B // NW
NVEC = N // L
U = 8

_NEG = -3.0e38


def _splat_f(x):
    return jnp.full((L,), x, jnp.float32)


CH = 4
CHN = N // CH


def _row_tau(row_v, cand_v, in_copies):
    ones = jnp.full((L,), 1, jnp.int32)
    zeros = jnp.full((L,), 0, jnp.int32)
    iota = lax.iota(jnp.int32, L)

    RU = N // U

    def sweep_body(i, carry):
        thp, acc, idxs = carry
        vs = [row_v[pl.ds((i * U + u) * L, L)] for u in range(U)]
        new_idxs = []
        for u in range(U):
            msk = vs[u] >= thp
            plsc.store_scatter(cand_v, [idxs[u]], vs[u], mask=msk)
            new_idxs.append(
                idxs[u] + jnp.where(msk, jnp.int32(L), jnp.int32(0)))
        bmax = vs
        while len(bmax) > 1:
            bmax = [jnp.maximum(a, b) for a, b in zip(bmax[::2], bmax[1::2])]
        acc = jnp.maximum(acc, bmax[0])
        thp = jnp.maximum(thp, bmax[0] - jnp.float32(1.0))
        return thp, acc, tuple(new_idxs)

    idxs0 = tuple(jnp.full((L,), u * RU, jnp.int32) + iota for u in range(U))
    carry = (_splat_f(_NEG), _splat_f(_NEG), idxs0)
    steps = NVEC // U // CH
    for c in range(CH):
        in_copies[c].wait()
        carry = lax.fori_loop(c * steps, (c + 1) * steps, sweep_body, carry)
    _, acc, idxs_v = carry
    cnts = [jnp.right_shift(idxs_v[u] - iota, 4) - jnp.int32(u * (RU >> 4))
            for u in range(U)]

    m = jnp.max(acc)
    cmax = cnts
    while len(cmax) > 1:
        cmax = [jnp.maximum(a, b) for a, b in zip(cmax[::2], cmax[1::2])]
    nv = jnp.max(cmax[0])

    def tau_from(sel_fn):
        def body(j, acc2):
            s, c = acc2
            for u in range(U):
                v = cand_v[pl.ds(u * RU + j * L, L)]
                sel = jnp.logical_and(cnts[u] > j, sel_fn(v))
                s = s + jnp.where(sel, v, jnp.float32(0.0))
                c = c + jnp.where(sel, ones, zeros)
            return s, c

        s, c = lax.fori_loop(0, nv, body, (_splat_f(0.0), zeros))
        cs = jnp.sum(c)
        sv = _splat_f(jnp.sum(s))
        cv = jnp.full((L,), cs).astype(jnp.float32)
        return (sv - jnp.float32(1.0)) / cv, cs

    t0, _ = tau_from(lambda v: v >= m)

    def w_cond(st):
        _, cprev, cnow, it = st
        return jnp.logical_and(cnow != cprev, it < jnp.int32(128))

    def w_body(st):
        t, _, cnow, it = st
        t2, c = tau_from(lambda v: v > t)
        return (t2, cnow, c, it + jnp.int32(1))

    tau, _, _, _ = lax.while_loop(
        w_cond, w_body, (t0, jnp.int32(-1), jnp.int32(-2), jnp.int32(0)))
    return tau


def _row_out(row_v, tau, out_hbm_row, out_sem):
    def out_body(i, _):
        for u in range(U):
            sl = pl.ds((i * U + u) * L, L)
            row_v[sl] = jnp.maximum(row_v[sl] - tau, jnp.float32(0.0))
        return 0

    steps = NVEC // U // CH
    copies = []
    for c in range(CH):
        lax.fori_loop(c * steps, (c + 1) * steps, out_body, 0)
        cp = pltpu.make_async_copy(
            row_v.at[pl.ds(c * CHN, CHN)],
            out_hbm_row.at[pl.ds(c * CHN, CHN)], out_sem)
        cp.start()
        copies.append(cp)
    return copies


@functools.partial(
    pl.kernel,
    out_type=jax.ShapeDtypeStruct((B, N), jnp.float32),
    mesh=plsc.VectorSubcoreMesh(core_axis_name="c", subcore_axis_name="s"),
    compiler_params=pltpu.CompilerParams(needs_layout_passes=False),
    scratch_types=[
        pltpu.VMEM((N,), jnp.float32),
        pltpu.VMEM((N,), jnp.float32),
        pltpu.VMEM((N,), jnp.float32),
        pltpu.SemaphoreType.DMA,
        pltpu.SemaphoreType.DMA,
        pltpu.SemaphoreType.DMA,
        pltpu.SemaphoreType.DMA,
    ],
)
def _sparsemax_sc(z_hbm, out_hbm, row0_v, row1_v, cand_v,
                  in0_sem, in1_sem, out0_sem, out1_sem):
    wid = lax.axis_index("s") * NC + lax.axis_index("c")
    r0 = wid * ROWS_PER_W
    r1 = r0 + 1
    in0 = [pltpu.make_async_copy(z_hbm.at[r0, pl.ds(c * CHN, CHN)],
                                 row0_v.at[pl.ds(c * CHN, CHN)], in0_sem)
           for c in range(CH)]
    in1 = [pltpu.make_async_copy(z_hbm.at[r1, pl.ds(c * CHN, CHN)],
                                 row1_v.at[pl.ds(c * CHN, CHN)], in1_sem)
           for c in range(CH)]
    for cp in in0 + in1:
        cp.start()
    tau0 = _row_tau(row0_v, cand_v, in0)
    out0 = _row_out(row0_v, tau0, out_hbm.at[r0], out0_sem)
    tau1 = _row_tau(row1_v, cand_v, in1)
    out1 = _row_out(row1_v, tau1, out_hbm.at[r1], out1_sem)
    for cp in out0 + out1:
        cp.wait()


def kernel(z):
    assert z.shape == (B, N) and z.dtype == jnp.float32
    return _sparsemax_sc(z)
